# baseline (speedup 1.0000x reference)
import functools

import jax
import jax.numpy as jnp
from jax import lax
from jax.experimental import pallas as pl
from jax.experimental.pallas import tpu as pltpu
from jax.experimental.pallas import tpu_sc as plsc

NUM_CLASSES = 100
IMAGE_SIZE = 65536
NC = 2
NS = 16
L = 16
NW = NC * NS
TC_COLS = 45056
SC_COLS = IMAGE_SIZE - TC_COLS
COLS_W = SC_COLS // NW
GROUPS = COLS_W // L
CB = 16
BLOCKS = [(i * CB, min(CB, NUM_CLASSES - i * CB))
          for i in range((NUM_CLASSES + CB - 1) // CB)]
SLOT = 128
BC = 8192


def _sc_partials(image_flat, basal):
    mesh = plsc.VectorSubcoreMesh(core_axis_name="c", subcore_axis_name="s")

    @functools.partial(
        pl.kernel,
        out_type=jax.ShapeDtypeStruct((NUM_CLASSES, NC * SLOT), jnp.float32),
        mesh=mesh,
        scratch_types=[
            pltpu.VMEM((COLS_W,), jnp.float32),
            pltpu.VMEM((2, CB, COLS_W), jnp.float32),
            pltpu.VMEM((NUM_CLASSES, SLOT), jnp.float32),
            pltpu.VMEM_SHARED((NUM_CLASSES, SLOT), jnp.float32),
            pltpu.VMEM((NUM_CLASSES,), jnp.int32),
            pltpu.SemaphoreType.DMA,
            pltpu.SemaphoreType.DMA,
        ],
    )
    def k(img_hbm, basal_hbm, out_hbm, mask_v, buf_v, acc_v, shr_v, idx_v,
          sem0, sem1):
        cid = lax.axis_index("c")
        sid = lax.axis_index("s")
        wid = sid * NC + cid
        base = TC_COLS + wid * COLS_W

        zeros = jnp.zeros((L,), jnp.float32)
        sems = (sem0, sem1)

        def start(t):
            row0, nrows = BLOCKS[t]
            return pltpu.async_copy(
                basal_hbm.at[pl.ds(row0, nrows), pl.ds(base, COLS_W)],
                buf_v.at[t % 2, pl.ds(0, nrows)],
                sems[t % 2],
            )

        handles = [start(0), start(1)]

        pltpu.sync_copy(img_hbm.at[pl.ds(base, COLS_W)], mask_v)

        def mk(g, carry):
            v = mask_v[pl.ds(g * L, L)]
            mask_v[pl.ds(g * L, L)] = jnp.where(v > 0.5, 1.0, 0.0)
            return carry

        lax.fori_loop(0, GROUPS, mk, 0)

        def compute(t):
            row0, nrows = BLOCKS[t]
            b = t % 2

            def body(g, accs):
                m = mask_v[pl.ds(g * L, L)]
                return tuple(
                    accs[j] + buf_v[b, j, pl.ds(g * L, L)] * m
                    for j in range(nrows)
                )

            accs = lax.fori_loop(
                0, GROUPS, body, (zeros,) * nrows, unroll=2)
            for j in range(nrows):
                acc_v[row0 + j, pl.ds(0, L)] = accs[j]

        for t in range(len(BLOCKS)):
            handles[t % 2].wait()
            compute(t)
            if t + 2 < len(BLOCKS):
                handles[t % 2] = start(t + 2)

        @pl.when(sid == 0)
        def _():
            pltpu.sync_copy(acc_v, shr_v)

        lane16 = lax.iota(jnp.int32, 16)
        for g, off in enumerate((0, 16, 32, 48, 64, 80, 84)):
            idx_v[pl.ds(off, L)] = lane16 + off

        plsc.subcore_barrier()

        @pl.when(sid != 0)
        def _():
            pltpu.sync_copy(acc_v, shr_v.at[idx_v], add=True)

        plsc.subcore_barrier()

        @pl.when(sid == 0)
        def _():
            pltpu.sync_copy(shr_v, out_hbm.at[:, pl.ds(cid * SLOT, SLOT)])

    return k(image_flat, basal)


def _tc_matvec(image, basal):
    RB = 40

    def body(img_ref, bas_ref, o_ref):
        m = (img_ref[...] > 0.5).astype(jnp.float32)
        o_ref[...] = jnp.sum(bas_ref[...] * m, axis=1, keepdims=True)

    return pl.pallas_call(
        body,
        grid=(pl.cdiv(NUM_CLASSES, RB),),
        in_specs=[
            pl.BlockSpec((1, TC_COLS), lambda i: (0, 0)),
            pl.BlockSpec((RB, TC_COLS), lambda i: (i, 0)),
        ],
        out_specs=pl.BlockSpec((RB, 1), lambda i: (i, 0)),
        out_shape=jax.ShapeDtypeStruct((NUM_CLASSES, 1), jnp.float32),
    )(image, basal)


def _tc_finish(partials, tc_part):
    def body(p_ref, q_ref, o_ref):
        p = p_ref[...]
        lane = lax.broadcasted_iota(jnp.int32, p.shape, 1)
        p = jnp.where(lane % SLOT < L, p, 0.0)
        s = jnp.sum(p, axis=1, keepdims=True) + q_ref[...]
        mx = jnp.max(s, axis=0, keepdims=True)
        idx = lax.broadcasted_iota(jnp.int32, s.shape, 0)
        cand = jnp.where(s >= mx, idx, jnp.int32(NUM_CLASSES))
        o_ref[0, 0] = jnp.min(cand)

    return pl.pallas_call(
        body,
        out_shape=jax.ShapeDtypeStruct((1, 1), jnp.int32),
        out_specs=pl.BlockSpec(memory_space=pltpu.SMEM),
    )(partials, tc_part)


def kernel(image, basal_synapses):
    img = image.reshape(IMAGE_SIZE)
    partials = _sc_partials(img, basal_synapses)
    tc_part = _tc_matvec(image, basal_synapses)
    label = _tc_finish(partials, tc_part)
    return label[0, 0]


# speedup vs baseline: 1.0014x; 1.0014x over previous
import functools

import jax
import jax.numpy as jnp
from jax import lax
from jax.experimental import pallas as pl
from jax.experimental.pallas import tpu as pltpu
from jax.experimental.pallas import tpu_sc as plsc

NUM_CLASSES = 100
IMAGE_SIZE = 65536
NC = 2
NS = 16
L = 16
NW = NC * NS
TC_COLS = 45056
SC_COLS = IMAGE_SIZE - TC_COLS
COLS_W = SC_COLS // NW
GROUPS = COLS_W // L
CB = 16
BLOCKS = [(i * CB, min(CB, NUM_CLASSES - i * CB))
          for i in range((NUM_CLASSES + CB - 1) // CB)]
SLOT = 128


def _sc_partials(image_flat, basal):
    mesh = plsc.VectorSubcoreMesh(core_axis_name="c", subcore_axis_name="s")

    @functools.partial(
        pl.kernel,
        out_type=jax.ShapeDtypeStruct((NUM_CLASSES, NC * SLOT), jnp.float32),
        mesh=mesh,
        scratch_types=[
            pltpu.VMEM((COLS_W,), jnp.float32),
            pltpu.VMEM((2, CB, COLS_W), jnp.float32),
            pltpu.VMEM((NUM_CLASSES, SLOT), jnp.float32),
            pltpu.VMEM_SHARED((NUM_CLASSES, SLOT), jnp.float32),
            pltpu.VMEM((NUM_CLASSES,), jnp.int32),
            pltpu.SemaphoreType.DMA,
            pltpu.SemaphoreType.DMA,
        ],
    )
    def k(img_hbm, basal_hbm, out_hbm, mask_v, buf_v, acc_v, shr_v, idx_v,
          sem0, sem1):
        cid = lax.axis_index("c")
        sid = lax.axis_index("s")
        wid = sid * NC + cid
        base = TC_COLS + wid * COLS_W

        zeros = jnp.zeros((L,), jnp.float32)
        sems = (sem0, sem1)

        def start(t):
            row0, nrows = BLOCKS[t]
            return pltpu.async_copy(
                basal_hbm.at[pl.ds(row0, nrows), pl.ds(base, COLS_W)],
                buf_v.at[t % 2, pl.ds(0, nrows)],
                sems[t % 2],
            )

        handles = [start(0), start(1)]

        pltpu.sync_copy(img_hbm.at[pl.ds(base, COLS_W)], mask_v)

        def mk(g, carry):
            v = mask_v[pl.ds(g * L, L)]
            mask_v[pl.ds(g * L, L)] = jnp.where(v > 0.5, 1.0, 0.0)
            return carry

        lax.fori_loop(0, GROUPS, mk, 0)

        def compute(t):
            row0, nrows = BLOCKS[t]
            b = t % 2

            def body(g, accs):
                m = mask_v[pl.ds(g * L, L)]
                return tuple(
                    accs[j] + buf_v[b, j, pl.ds(g * L, L)] * m
                    for j in range(nrows)
                )

            accs = lax.fori_loop(
                0, GROUPS, body, (zeros,) * nrows, unroll=2)
            for j in range(nrows):
                acc_v[row0 + j, pl.ds(0, L)] = accs[j]

        for t in range(len(BLOCKS)):
            handles[t % 2].wait()
            compute(t)
            if t + 2 < len(BLOCKS):
                handles[t % 2] = start(t + 2)

        @pl.when(sid == 0)
        def _():
            pltpu.sync_copy(acc_v, shr_v)

        lane16 = lax.iota(jnp.int32, 16)
        for g, off in enumerate((0, 16, 32, 48, 64, 80, 84)):
            idx_v[pl.ds(off, L)] = lane16 + off

        plsc.subcore_barrier()

        @pl.when(sid != 0)
        def _():
            pltpu.sync_copy(acc_v, shr_v.at[idx_v], add=True)

        plsc.subcore_barrier()

        @pl.when(sid == 0)
        def _():
            pltpu.sync_copy(shr_v, out_hbm.at[:, pl.ds(cid * SLOT, SLOT)])

    return k(image_flat, basal)


def _tc_matvec(image, basal):
    RB = 32

    def body(img_ref, bas_ref, o_ref):
        m = (img_ref[...] > 0.5).astype(jnp.float32)
        o_ref[...] = jnp.sum(bas_ref[...] * m, axis=1, keepdims=True)

    return pl.pallas_call(
        body,
        grid=(pl.cdiv(NUM_CLASSES, RB),),
        in_specs=[
            pl.BlockSpec((1, TC_COLS), lambda i: (0, 0)),
            pl.BlockSpec((RB, TC_COLS), lambda i: (i, 0)),
        ],
        out_specs=pl.BlockSpec((RB, 1), lambda i: (i, 0)),
        out_shape=jax.ShapeDtypeStruct((NUM_CLASSES, 1), jnp.float32),
    )(image, basal)


def _tc_finish(partials, tc_part):
    def body(p_ref, q_ref, o_ref):
        p = p_ref[...]
        lane = lax.broadcasted_iota(jnp.int32, p.shape, 1)
        p = jnp.where(lane % SLOT < L, p, 0.0)
        s = jnp.sum(p, axis=1, keepdims=True) + q_ref[...]
        mx = jnp.max(s, axis=0, keepdims=True)
        idx = lax.broadcasted_iota(jnp.int32, s.shape, 0)
        cand = jnp.where(s >= mx, idx, jnp.int32(NUM_CLASSES))
        o_ref[0, 0] = jnp.min(cand)

    return pl.pallas_call(
        body,
        out_shape=jax.ShapeDtypeStruct((1, 1), jnp.int32),
        out_specs=pl.BlockSpec(memory_space=pltpu.SMEM),
    )(partials, tc_part)


def kernel(image, basal_synapses):
    img = image.reshape(IMAGE_SIZE)
    partials = _sc_partials(img, basal_synapses)
    tc_part = _tc_matvec(image, basal_synapses)
    label = _tc_finish(partials, tc_part)
    return label[0, 0]
